# Initial kernel scaffold; baseline (speedup 1.0000x reference)
import jax
import jax.numpy as jnp
from jax.experimental import pallas as pl


def kernel(x, edge_index, batch, cluster_feats, W1, b1, W2, b2, lin_W, lin_b):
    raise NotImplementedError("write your pallas kernel here")


# speedup vs baseline: 17.0396x; 17.0396x over previous
import functools

import jax
import jax.numpy as jnp
from jax import lax
from jax.experimental import pallas as pl
from jax.experimental.pallas import tpu as pltpu
from jax.experimental.pallas import tpu_sc as plsc

N = 10000
NPAD = 10240
E = 320000
D_IN = 128
D_HID = 256
HALF = 128
B = 64
EXTRA = 16

NC = 2
NS = 16
CH = 125
ROWS_E = E // CH
RPW_DEG = ROWS_E // (NC * NS)
RPC_SPMM = ROWS_E // NS
RPS = NPAD // NS
ZCH = 128
GR = 16
RB = 256
GRID = NPAD // RB


def _deg_body(dst2d, zeros80, ones128, out, acc, dst_v, ones_v, stage_v):
    c = lax.axis_index("c")
    s = lax.axis_index("s")
    pltpu.sync_copy(zeros80, stage_v)
    for t in range(RPS // 80):
        pltpu.sync_copy(stage_v, acc.at[pl.ds(s * RPS + t * 80, 80)])
    pltpu.sync_copy(ones128, ones_v)
    wid = s * NC + c
    pltpu.sync_copy(dst2d.at[pl.ds(wid * RPW_DEG, RPW_DEG)], dst_v)
    plsc.subcore_barrier()

    def body(j, carry):
        pltpu.sync_copy(ones_v, acc.at[dst_v.at[j]], add=True)
        return carry

    lax.fori_loop(0, RPW_DEG, body, 0)
    plsc.subcore_barrier()
    for t in range(RPS // 80):
        sl = pl.ds(s * RPS + t * 80, 80)
        pltpu.sync_copy(acc.at[sl], stage_v)
        pltpu.sync_copy(stage_v, out.at[c, sl])


def _deg_call(dst2d, zeros80, ones128):
    mesh = plsc.VectorSubcoreMesh(core_axis_name="c", subcore_axis_name="s")
    return pl.kernel(
        _deg_body,
        out_type=jax.ShapeDtypeStruct((NC, NPAD, HALF), jnp.float32),
        mesh=mesh,
        scratch_types=[
            pltpu.VMEM_SHARED((NPAD, HALF), jnp.float32),
            pltpu.VMEM((RPW_DEG, CH), jnp.int32),
            pltpu.VMEM((CH, HALF), jnp.float32),
            pltpu.VMEM((80, HALF), jnp.float32),
        ],
    )(dst2d, zeros80, ones128)


def _spmm_body(s_pair, src2d, dst2d, zeros80, zdrain, out,
               acc, src_v, dst_v, rows0, rows1, gsem):
    c = lax.axis_index("c")
    s = lax.axis_index("s")
    z80 = rows0.at[pl.ds(0, 80)]
    pltpu.sync_copy(zeros80, z80)
    for t in range(RPS // 80):
        pltpu.sync_copy(z80, acc.at[pl.ds(s * RPS + t * 80, 80)])
    base = s * RPC_SPMM
    plsc.subcore_barrier()

    table = s_pair.at[c]

    def gbody(g, carry):
        pltpu.sync_copy(src2d.at[pl.ds(base + g * GR, GR)], src_v)
        pltpu.sync_copy(dst2d.at[pl.ds(base + g * GR, GR)], dst_v)
        pltpu.async_copy(table.at[src_v.at[0]], rows0, gsem)

        def body(jj, carry2):
            j0 = jj * 2
            pltpu.make_async_copy(zdrain, rows0, gsem).wait()
            pltpu.async_copy(table.at[src_v.at[j0 + 1]], rows1, gsem)
            pltpu.sync_copy(rows0, acc.at[dst_v.at[j0]], add=True)
            pltpu.make_async_copy(zdrain, rows1, gsem).wait()

            @pl.when(j0 + 2 < GR)
            def _():
                pltpu.async_copy(table.at[src_v.at[j0 + 2]], rows0, gsem)

            pltpu.sync_copy(rows1, acc.at[dst_v.at[j0 + 1]], add=True)
            return carry2

        lax.fori_loop(0, GR // 2, body, 0)
        return carry

    lax.fori_loop(0, RPC_SPMM // GR, gbody, 0)
    plsc.subcore_barrier()
    for t in range(RPS // 80):
        sl = pl.ds(s * RPS + t * 80, 80)
        pltpu.sync_copy(acc.at[sl], z80)
        pltpu.sync_copy(z80, out.at[c, sl])


def _spmm_call(s_pair, src2d, dst2d, zeros80, zdrain):
    mesh = plsc.VectorSubcoreMesh(core_axis_name="c", subcore_axis_name="s")
    return pl.kernel(
        _spmm_body,
        out_type=jax.ShapeDtypeStruct((NC, NPAD, HALF), jnp.float32),
        mesh=mesh,
        scratch_types=[
            pltpu.VMEM_SHARED((NPAD, HALF), jnp.float32),
            pltpu.VMEM((GR, CH), jnp.int32),
            pltpu.VMEM((GR, CH), jnp.int32),
            pltpu.VMEM((CH, HALF), jnp.float32),
            pltpu.VMEM((CH, HALF), jnp.float32),
            pltpu.SemaphoreType.DMA,
        ],
    )(s_pair, src2d, dst2d, zeros80, zdrain)


def _tca_body(x_ref, w1_ref, deg_ref, s_ref, dinv_ref):
    deg = deg_ref[0, :, 0:1] + deg_ref[1, :, 0:1] + 1.0
    dinv = lax.rsqrt(deg)
    z = jnp.dot(x_ref[...], w1_ref[...], preferred_element_type=jnp.float32)
    sarr = z * dinv
    s_ref[0] = sarr[:, :HALF]
    s_ref[1] = sarr[:, HALF:]
    dinv_ref[...] = dinv


def _tca_call(xp, W1, deg2):
    return pl.pallas_call(
        _tca_body,
        grid=(GRID,),
        in_specs=[
            pl.BlockSpec((RB, D_IN), lambda i: (i, 0)),
            pl.BlockSpec((D_IN, D_HID), lambda i: (0, 0)),
            pl.BlockSpec((NC, RB, HALF), lambda i: (0, i, 0)),
        ],
        out_specs=[
            pl.BlockSpec((NC, RB, HALF), lambda i: (0, i, 0)),
            pl.BlockSpec((RB, 1), lambda i: (i, 0)),
        ],
        out_shape=[
            jax.ShapeDtypeStruct((NC, NPAD, HALF), jnp.float32),
            jax.ShapeDtypeStruct((NPAD, 1), jnp.float32),
        ],
    )(xp, W1, deg2)


def _tcb_body(agg_ref, s_ref, dinv_ref, b1_ref, w2_ref, out_ref):
    dinv = dinv_ref[...]
    h1a = jnp.maximum((agg_ref[0] + s_ref[0]) * dinv + b1_ref[:, :HALF], 0.0)
    h1b = jnp.maximum((agg_ref[1] + s_ref[1]) * dinv + b1_ref[:, HALF:], 0.0)
    h1 = jnp.concatenate([h1a, h1b], axis=1)
    z2 = jnp.dot(h1, w2_ref[...], preferred_element_type=jnp.float32)
    s2 = z2 * dinv
    out_ref[0] = s2[:, :HALF]
    out_ref[1] = s2[:, HALF:]


def _tcb_call(agg1, s1_pair, dinv, b1r, W2):
    return pl.pallas_call(
        _tcb_body,
        grid=(GRID,),
        in_specs=[
            pl.BlockSpec((NC, RB, HALF), lambda i: (0, i, 0)),
            pl.BlockSpec((NC, RB, HALF), lambda i: (0, i, 0)),
            pl.BlockSpec((RB, 1), lambda i: (i, 0)),
            pl.BlockSpec((1, D_HID), lambda i: (0, 0)),
            pl.BlockSpec((D_HID, D_HID), lambda i: (0, 0)),
        ],
        out_specs=pl.BlockSpec((NC, RB, HALF), lambda i: (0, i, 0)),
        out_shape=jax.ShapeDtypeStruct((NC, NPAD, HALF), jnp.float32),
    )(agg1, s1_pair, dinv, b1r, W2)


def _tcc_body(agg_ref, s_ref, dinv_ref, b2_ref, batch_ref, cf_ref,
              lwg_ref, lwc_ref, lb_ref, out_ref, acc_sum, acc_cnt):
    i = pl.program_id(0)

    @pl.when(i == 0)
    def _():
        acc_sum[...] = jnp.zeros_like(acc_sum)
        acc_cnt[...] = jnp.zeros_like(acc_cnt)

    dinv = dinv_ref[...]
    h2a = jnp.maximum((agg_ref[0] + s_ref[0]) * dinv + b2_ref[:, :HALF], 0.0)
    h2b = jnp.maximum((agg_ref[1] + s_ref[1]) * dinv + b2_ref[:, HALF:], 0.0)
    h2 = jnp.concatenate([h2a, h2b], axis=1)
    bb = batch_ref[...]
    ids = lax.broadcasted_iota(jnp.int32, (RB, B), 1)
    pt = (bb == ids).astype(jnp.float32)
    acc_sum[...] += lax.dot_general(pt, h2, (((0,), (0,)), ((), ())),
                                    preferred_element_type=jnp.float32)
    acc_cnt[...] += lax.dot_general(pt, jnp.ones((RB, 1), jnp.float32),
                                    (((0,), (0,)), ((), ())),
                                    preferred_element_type=jnp.float32)

    @pl.when(i == pl.num_programs(0) - 1)
    def _():
        g = acc_sum[...] / jnp.maximum(acc_cnt[...], 1.0)
        res = (jnp.dot(g, lwg_ref[...], preferred_element_type=jnp.float32)
               + jnp.dot(cf_ref[...], lwc_ref[...],
                         preferred_element_type=jnp.float32)
               + lb_ref[...])
        out_ref[...] = res


def _tcc_call(agg2, s2_pair, dinv, b2r, batch_p, cf, lwg, lwc, lbr):
    return pl.pallas_call(
        _tcc_body,
        grid=(GRID,),
        in_specs=[
            pl.BlockSpec((NC, RB, HALF), lambda i: (0, i, 0)),
            pl.BlockSpec((NC, RB, HALF), lambda i: (0, i, 0)),
            pl.BlockSpec((RB, 1), lambda i: (i, 0)),
            pl.BlockSpec((1, D_HID), lambda i: (0, 0)),
            pl.BlockSpec((RB, 1), lambda i: (i, 0)),
            pl.BlockSpec((B, EXTRA), lambda i: (0, 0)),
            pl.BlockSpec((D_HID, 2), lambda i: (0, 0)),
            pl.BlockSpec((EXTRA, 2), lambda i: (0, 0)),
            pl.BlockSpec((1, 2), lambda i: (0, 0)),
        ],
        out_specs=pl.BlockSpec((B, 2), lambda i: (0, 0)),
        out_shape=jax.ShapeDtypeStruct((B, 2), jnp.float32),
        scratch_shapes=[
            pltpu.VMEM((B, D_HID), jnp.float32),
            pltpu.VMEM((B, 1), jnp.float32),
        ],
    )(agg2, s2_pair, dinv, b2r, batch_p, cf, lwg, lwc, lbr)


def kernel(x, edge_index, batch, cluster_feats, W1, b1, W2, b2, lin_W, lin_b):
    f32 = jnp.float32
    src2d = edge_index[0].reshape(ROWS_E, CH)
    dst2d = edge_index[1].reshape(ROWS_E, CH)
    xp = jnp.zeros((NPAD, D_IN), f32).at[:N].set(x)
    batch_p = jnp.concatenate(
        [batch, jnp.full((NPAD - N,), B, jnp.int32)]).reshape(NPAD, 1)
    ones128 = jnp.ones((CH, HALF), f32)
    zeros80 = jnp.zeros((80, HALF), f32)
    zdrain = jnp.zeros((CH, HALF), f32)
    b1r = b1.reshape(1, D_HID)
    b2r = b2.reshape(1, D_HID)
    lwg = lin_W[:D_HID]
    lwc = lin_W[D_HID:]
    lbr = lin_b.reshape(1, 2)

    deg2 = _deg_call(dst2d, zeros80, ones128)
    s1_pair, dinv = _tca_call(xp, W1, deg2)
    agg1 = _spmm_call(s1_pair, src2d, dst2d, zeros80, zdrain)
    s2_pair = _tcb_call(agg1, s1_pair, dinv, b1r, W2)
    agg2 = _spmm_call(s2_pair, src2d, dst2d, zeros80, zdrain)
    return _tcc_call(agg2, s2_pair, dinv, b2r, batch_p, cluster_feats,
                     lwg, lwc, lbr)


# speedup vs baseline: 20.4687x; 1.2012x over previous
import functools

import jax
import jax.numpy as jnp
from jax import lax
from jax.experimental import pallas as pl
from jax.experimental.pallas import tpu as pltpu
from jax.experimental.pallas import tpu_sc as plsc

N = 10000
NPAD = 10240
E = 320000
D_IN = 128
D_HID = 256
HALF = 128
B = 64
EXTRA = 16

NC = 2
NS = 16
CH = 125
ROWS_E = E // CH
RPW_DEG = ROWS_E // (NC * NS)
RPC_SPMM = ROWS_E // NS
RPS = NPAD // NS
ZCH = 128
GR = 16
RB = 256
GRID = NPAD // RB


def _deg_body(dst2d, zeros80, ones128, out, acc, dst_v, ones_v, stage_v):
    c = lax.axis_index("c")
    s = lax.axis_index("s")
    pltpu.sync_copy(zeros80, stage_v)
    for t in range(RPS // 80):
        pltpu.sync_copy(stage_v, acc.at[pl.ds(s * RPS + t * 80, 80)])
    pltpu.sync_copy(ones128, ones_v)
    wid = s * NC + c
    pltpu.sync_copy(dst2d.at[pl.ds(wid * RPW_DEG, RPW_DEG)], dst_v)
    plsc.subcore_barrier()

    def body(j, carry):
        pltpu.sync_copy(ones_v, acc.at[dst_v.at[j]], add=True)
        return carry

    lax.fori_loop(0, RPW_DEG, body, 0)
    plsc.subcore_barrier()
    for t in range(RPS // 80):
        sl = pl.ds(s * RPS + t * 80, 80)
        pltpu.sync_copy(acc.at[sl], stage_v)
        pltpu.sync_copy(stage_v, out.at[c, sl])


def _deg_call(dst2d, zeros80, ones128):
    mesh = plsc.VectorSubcoreMesh(core_axis_name="c", subcore_axis_name="s")
    return pl.kernel(
        _deg_body,
        out_type=jax.ShapeDtypeStruct((NC, NPAD, HALF), jnp.float32),
        mesh=mesh,
        scratch_types=[
            pltpu.VMEM_SHARED((NPAD, HALF), jnp.float32),
            pltpu.VMEM((RPW_DEG, CH), jnp.int32),
            pltpu.VMEM((CH, HALF), jnp.float32),
            pltpu.VMEM((80, HALF), jnp.float32),
        ],
    )(dst2d, zeros80, ones128)


def _spmm_body(s_pair, src2d, dst2d, zeros80, zdrain, out,
               acc, src_v, dst_v, rows0, rows1, gsem):
    c = lax.axis_index("c")
    s = lax.axis_index("s")
    z80 = rows0.at[pl.ds(0, 80)]
    pltpu.sync_copy(zeros80, z80)
    for t in range(RPS // 80):
        pltpu.sync_copy(z80, acc.at[pl.ds(s * RPS + t * 80, 80)])
    base = s * RPC_SPMM
    plsc.subcore_barrier()

    table = s_pair.at[c]

    def gbody(g, carry):
        pltpu.sync_copy(src2d.at[pl.ds(base + g * GR, GR)], src_v)
        pltpu.sync_copy(dst2d.at[pl.ds(base + g * GR, GR)], dst_v)
        pltpu.async_copy(table.at[src_v.at[0]], rows0, gsem)

        def body(jj, carry2):
            j0 = jj * 2
            pltpu.make_async_copy(zdrain, rows0, gsem).wait()
            pltpu.async_copy(table.at[src_v.at[j0 + 1]], rows1, gsem)
            pltpu.sync_copy(rows0, acc.at[dst_v.at[j0]], add=True)
            pltpu.make_async_copy(zdrain, rows1, gsem).wait()

            @pl.when(j0 + 2 < GR)
            def _():
                pltpu.async_copy(table.at[src_v.at[j0 + 2]], rows0, gsem)

            pltpu.sync_copy(rows1, acc.at[dst_v.at[j0 + 1]], add=True)
            return carry2

        lax.fori_loop(0, GR // 2, body, 0)
        return carry

    lax.fori_loop(0, RPC_SPMM // GR, gbody, 0)
    plsc.subcore_barrier()
    for t in range(RPS // 80):
        sl = pl.ds(s * RPS + t * 80, 80)
        pltpu.sync_copy(acc.at[sl], z80)
        pltpu.sync_copy(z80, out.at[c, sl])


def _spmm_x_body(y, src2d, dst2d, zeros80, zdrain, out,
                 acc, src_v, dst_v, rows0, rows1, gsem):
    c = lax.axis_index("c")
    s = lax.axis_index("s")
    z80 = rows0.at[pl.ds(0, 80)]
    pltpu.sync_copy(zeros80, z80)
    for t in range(RPS // 80):
        pltpu.sync_copy(z80, acc.at[pl.ds(s * RPS + t * 80, 80)])
    base = (c * NS + s) * RPW_DEG
    plsc.subcore_barrier()

    def gbody(g, carry):
        pltpu.sync_copy(src2d.at[pl.ds(base + g * GR, GR)], src_v)
        pltpu.sync_copy(dst2d.at[pl.ds(base + g * GR, GR)], dst_v)
        pltpu.async_copy(y.at[src_v.at[0]], rows0, gsem)

        def body(jj, carry2):
            j0 = jj * 2
            pltpu.make_async_copy(zdrain, rows0, gsem).wait()
            pltpu.async_copy(y.at[src_v.at[j0 + 1]], rows1, gsem)
            pltpu.sync_copy(rows0, acc.at[dst_v.at[j0]], add=True)
            pltpu.make_async_copy(zdrain, rows1, gsem).wait()

            @pl.when(j0 + 2 < GR)
            def _():
                pltpu.async_copy(y.at[src_v.at[j0 + 2]], rows0, gsem)

            pltpu.sync_copy(rows1, acc.at[dst_v.at[j0 + 1]], add=True)
            return carry2

        lax.fori_loop(0, GR // 2, body, 0)
        return carry

    lax.fori_loop(0, RPW_DEG // GR, gbody, 0)
    plsc.subcore_barrier()
    for t in range(RPS // 80):
        sl = pl.ds(s * RPS + t * 80, 80)
        pltpu.sync_copy(acc.at[sl], z80)
        pltpu.sync_copy(z80, out.at[c, sl])


def _spmm_x_call(y, src2d, dst2d, zeros80, zdrain):
    mesh = plsc.VectorSubcoreMesh(core_axis_name="c", subcore_axis_name="s")
    return pl.kernel(
        _spmm_x_body,
        out_type=jax.ShapeDtypeStruct((NC, NPAD, HALF), jnp.float32),
        mesh=mesh,
        scratch_types=[
            pltpu.VMEM_SHARED((NPAD, HALF), jnp.float32),
            pltpu.VMEM((GR, CH), jnp.int32),
            pltpu.VMEM((GR, CH), jnp.int32),
            pltpu.VMEM((CH, HALF), jnp.float32),
            pltpu.VMEM((CH, HALF), jnp.float32),
            pltpu.SemaphoreType.DMA,
        ],
    )(y, src2d, dst2d, zeros80, zdrain)


def _spmm_call(s_pair, src2d, dst2d, zeros80, zdrain):
    mesh = plsc.VectorSubcoreMesh(core_axis_name="c", subcore_axis_name="s")
    return pl.kernel(
        _spmm_body,
        out_type=jax.ShapeDtypeStruct((NC, NPAD, HALF), jnp.float32),
        mesh=mesh,
        scratch_types=[
            pltpu.VMEM_SHARED((NPAD, HALF), jnp.float32),
            pltpu.VMEM((GR, CH), jnp.int32),
            pltpu.VMEM((GR, CH), jnp.int32),
            pltpu.VMEM((CH, HALF), jnp.float32),
            pltpu.VMEM((CH, HALF), jnp.float32),
            pltpu.SemaphoreType.DMA,
        ],
    )(s_pair, src2d, dst2d, zeros80, zdrain)


def _tca_body(x_ref, deg_ref, y_ref, dinv_ref):
    deg = deg_ref[0, :, 0:1] + deg_ref[1, :, 0:1] + 1.0
    dinv = lax.rsqrt(deg)
    y_ref[...] = x_ref[...] * dinv
    dinv_ref[...] = dinv


def _tca_call(xp, deg2):
    return pl.pallas_call(
        _tca_body,
        grid=(GRID,),
        in_specs=[
            pl.BlockSpec((RB, D_IN), lambda i: (i, 0)),
            pl.BlockSpec((NC, RB, HALF), lambda i: (0, i, 0)),
        ],
        out_specs=[
            pl.BlockSpec((RB, D_IN), lambda i: (i, 0)),
            pl.BlockSpec((RB, 1), lambda i: (i, 0)),
        ],
        out_shape=[
            jax.ShapeDtypeStruct((NPAD, D_IN), jnp.float32),
            jax.ShapeDtypeStruct((NPAD, 1), jnp.float32),
        ],
    )(xp, deg2)


def _tcb_body(aggx_ref, y_ref, dinv_ref, b1_ref, w1_ref, w2_ref, out_ref):
    dinv = dinv_ref[...]
    u = aggx_ref[0] + aggx_ref[1] + y_ref[...]
    z1 = jnp.dot(u, w1_ref[...], preferred_element_type=jnp.float32)
    h1 = jnp.maximum(z1 * dinv + b1_ref[...], 0.0)
    z2 = jnp.dot(h1, w2_ref[...], preferred_element_type=jnp.float32)
    s2 = z2 * dinv
    out_ref[0] = s2[:, :HALF]
    out_ref[1] = s2[:, HALF:]


def _tcb_call(aggx, y, dinv, b1r, W1, W2):
    return pl.pallas_call(
        _tcb_body,
        grid=(GRID,),
        in_specs=[
            pl.BlockSpec((NC, RB, HALF), lambda i: (0, i, 0)),
            pl.BlockSpec((RB, D_IN), lambda i: (i, 0)),
            pl.BlockSpec((RB, 1), lambda i: (i, 0)),
            pl.BlockSpec((1, D_HID), lambda i: (0, 0)),
            pl.BlockSpec((D_IN, D_HID), lambda i: (0, 0)),
            pl.BlockSpec((D_HID, D_HID), lambda i: (0, 0)),
        ],
        out_specs=pl.BlockSpec((NC, RB, HALF), lambda i: (0, i, 0)),
        out_shape=jax.ShapeDtypeStruct((NC, NPAD, HALF), jnp.float32),
    )(aggx, y, dinv, b1r, W1, W2)


def _tcc_body(agg_ref, s_ref, dinv_ref, b2_ref, batch_ref, cf_ref,
              lwg_ref, lwc_ref, lb_ref, out_ref, acc_sum, acc_cnt):
    i = pl.program_id(0)

    @pl.when(i == 0)
    def _():
        acc_sum[...] = jnp.zeros_like(acc_sum)
        acc_cnt[...] = jnp.zeros_like(acc_cnt)

    dinv = dinv_ref[...]
    h2a = jnp.maximum((agg_ref[0] + s_ref[0]) * dinv + b2_ref[:, :HALF], 0.0)
    h2b = jnp.maximum((agg_ref[1] + s_ref[1]) * dinv + b2_ref[:, HALF:], 0.0)
    h2 = jnp.concatenate([h2a, h2b], axis=1)
    bb = batch_ref[...]
    ids = lax.broadcasted_iota(jnp.int32, (RB, B), 1)
    pt = (bb == ids).astype(jnp.float32)
    acc_sum[...] += lax.dot_general(pt, h2, (((0,), (0,)), ((), ())),
                                    preferred_element_type=jnp.float32)
    acc_cnt[...] += lax.dot_general(pt, jnp.ones((RB, 1), jnp.float32),
                                    (((0,), (0,)), ((), ())),
                                    preferred_element_type=jnp.float32)

    @pl.when(i == pl.num_programs(0) - 1)
    def _():
        g = acc_sum[...] / jnp.maximum(acc_cnt[...], 1.0)
        res = (jnp.dot(g, lwg_ref[...], preferred_element_type=jnp.float32)
               + jnp.dot(cf_ref[...], lwc_ref[...],
                         preferred_element_type=jnp.float32)
               + lb_ref[...])
        out_ref[...] = res


def _tcc_call(agg2, s2_pair, dinv, b2r, batch_p, cf, lwg, lwc, lbr):
    return pl.pallas_call(
        _tcc_body,
        grid=(GRID,),
        in_specs=[
            pl.BlockSpec((NC, RB, HALF), lambda i: (0, i, 0)),
            pl.BlockSpec((NC, RB, HALF), lambda i: (0, i, 0)),
            pl.BlockSpec((RB, 1), lambda i: (i, 0)),
            pl.BlockSpec((1, D_HID), lambda i: (0, 0)),
            pl.BlockSpec((RB, 1), lambda i: (i, 0)),
            pl.BlockSpec((B, EXTRA), lambda i: (0, 0)),
            pl.BlockSpec((D_HID, 2), lambda i: (0, 0)),
            pl.BlockSpec((EXTRA, 2), lambda i: (0, 0)),
            pl.BlockSpec((1, 2), lambda i: (0, 0)),
        ],
        out_specs=pl.BlockSpec((B, 2), lambda i: (0, 0)),
        out_shape=jax.ShapeDtypeStruct((B, 2), jnp.float32),
        scratch_shapes=[
            pltpu.VMEM((B, D_HID), jnp.float32),
            pltpu.VMEM((B, 1), jnp.float32),
        ],
    )(agg2, s2_pair, dinv, b2r, batch_p, cf, lwg, lwc, lbr)


def kernel(x, edge_index, batch, cluster_feats, W1, b1, W2, b2, lin_W, lin_b):
    f32 = jnp.float32
    src2d = edge_index[0].reshape(ROWS_E, CH)
    dst2d = edge_index[1].reshape(ROWS_E, CH)
    xp = jnp.zeros((NPAD, D_IN), f32).at[:N].set(x)
    batch_p = jnp.concatenate(
        [batch, jnp.full((NPAD - N,), B, jnp.int32)]).reshape(NPAD, 1)
    ones128 = jnp.ones((CH, HALF), f32)
    zeros80 = jnp.zeros((80, HALF), f32)
    zdrain = jnp.zeros((CH, HALF), f32)
    b1r = b1.reshape(1, D_HID)
    b2r = b2.reshape(1, D_HID)
    lwg = lin_W[:D_HID]
    lwc = lin_W[D_HID:]
    lbr = lin_b.reshape(1, 2)

    deg2 = _deg_call(dst2d, zeros80, ones128)
    y, dinv = _tca_call(xp, deg2)
    aggx = _spmm_x_call(y, src2d, dst2d, zeros80, zdrain)
    s2_pair = _tcb_call(aggx, y, dinv, b1r, W1, W2)
    agg2 = _spmm_call(s2_pair, src2d, dst2d, zeros80, zdrain)
    return _tcc_call(agg2, s2_pair, dinv, b2r, batch_p, cluster_feats,
                     lwg, lwc, lbr)


# speedup vs baseline: 21.3856x; 1.0448x over previous
import functools

import jax
import jax.numpy as jnp
from jax import lax
from jax.experimental import pallas as pl
from jax.experimental.pallas import tpu as pltpu
from jax.experimental.pallas import tpu_sc as plsc

N = 10000
NPAD = 10240
E = 320000
D_IN = 128
D_HID = 256
HALF = 128
B = 64
EXTRA = 16

NC = 2
NS = 16
CH = 125
ROWS_E = E // CH
RPW_DEG = ROWS_E // (NC * NS)
RPS = NPAD // NS
SCH = 50
SROWS = E // SCH
SGR = 40
NBUF = 4
RB = 256
GRID = NPAD // RB


def _deg_body(dst2d, zeros80, ones128, out, acc, dst_v, ones_v, stage_v):
    c = lax.axis_index("c")
    s = lax.axis_index("s")
    pltpu.sync_copy(zeros80, stage_v)
    for t in range(RPS // 80):
        pltpu.sync_copy(stage_v, acc.at[pl.ds(s * RPS + t * 80, 80)])
    pltpu.sync_copy(ones128, ones_v)
    wid = s * NC + c
    pltpu.sync_copy(dst2d.at[pl.ds(wid * RPW_DEG, RPW_DEG)], dst_v)
    plsc.subcore_barrier()

    def body(j, carry):
        pltpu.sync_copy(ones_v, acc.at[dst_v.at[j]], add=True)
        return carry

    lax.fori_loop(0, RPW_DEG, body, 0)
    plsc.subcore_barrier()
    for t in range(RPS // 80):
        sl = pl.ds(s * RPS + t * 80, 80)
        pltpu.sync_copy(acc.at[sl], stage_v)
        pltpu.sync_copy(stage_v, out.at[c, sl])


def _deg_call(dst2d, zeros80, ones128):
    mesh = plsc.VectorSubcoreMesh(core_axis_name="c", subcore_axis_name="s")
    return pl.kernel(
        _deg_body,
        out_type=jax.ShapeDtypeStruct((NC, NPAD, HALF), jnp.float32),
        mesh=mesh,
        scratch_types=[
            pltpu.VMEM_SHARED((NPAD, HALF), jnp.float32),
            pltpu.VMEM((RPW_DEG, CH), jnp.int32),
            pltpu.VMEM((CH, HALF), jnp.float32),
            pltpu.VMEM((80, HALF), jnp.float32),
        ],
    )(dst2d, zeros80, ones128)


def _make_spmm_body(edge_split):
    rows_per_worker = SROWS // (NC * NS) if edge_split else SROWS // NS
    ngroups = rows_per_worker // SGR

    def body(table_arg, src2d, dst2d, zeros80, zdrain, out,
             acc, src_v, dst_v, b0, b1, b2, b3, stage_v, gsem, ssem):
        c = lax.axis_index("c")
        s = lax.axis_index("s")
        bufs = (b0, b1, b2, b3)
        pltpu.sync_copy(zeros80, stage_v)
        for t in range(RPS // 80):
            pltpu.sync_copy(stage_v, acc.at[pl.ds(s * RPS + t * 80, 80)])
        if edge_split:
            base = (c * NS + s) * rows_per_worker
            table = table_arg
        else:
            base = s * rows_per_worker
            table = table_arg.at[c]
        plsc.subcore_barrier()

        def gbody(g, carry):
            pltpu.sync_copy(src2d.at[pl.ds(base + g * SGR, SGR)], src_v)
            pltpu.sync_copy(dst2d.at[pl.ds(base + g * SGR, SGR)], dst_v)
            pltpu.async_copy(table.at[src_v.at[0]], b0, gsem)
            pltpu.async_copy(table.at[src_v.at[1]], b1, gsem)

            def cbody(jj, carry2):
                j0 = jj * NBUF
                for k in range(NBUF):
                    j = j0 + k
                    buf = bufs[k]
                    pltpu.make_async_copy(zdrain, buf, gsem).wait()

                    @pl.when(j >= 2)
                    def _():
                        pltpu.make_async_copy(zdrain, bufs[(k + 2) % NBUF],
                                              ssem).wait()

                    pltpu.async_copy(buf, acc.at[dst_v.at[j]], ssem, add=True)

                    @pl.when(j + 2 < SGR)
                    def _():
                        pltpu.async_copy(table.at[src_v.at[j + 2]],
                                         bufs[(k + 2) % NBUF], gsem)

                return carry2

            lax.fori_loop(0, SGR // NBUF, cbody, 0)
            pltpu.make_async_copy(zdrain, b2, ssem).wait()
            pltpu.make_async_copy(zdrain, b3, ssem).wait()
            return carry

        lax.fori_loop(0, ngroups, gbody, 0)
        plsc.subcore_barrier()
        for t in range(RPS // 80):
            sl = pl.ds(s * RPS + t * 80, 80)
            pltpu.sync_copy(acc.at[sl], stage_v)
            pltpu.sync_copy(stage_v, out.at[c, sl])

    return body


_spmm_body = _make_spmm_body(edge_split=False)
_spmm_x_body = _make_spmm_body(edge_split=True)


def _spmm_scratch():
    return [
        pltpu.VMEM_SHARED((NPAD, HALF), jnp.float32),
        pltpu.VMEM((SGR, SCH), jnp.int32),
        pltpu.VMEM((SGR, SCH), jnp.int32),
        pltpu.VMEM((SCH, HALF), jnp.float32),
        pltpu.VMEM((SCH, HALF), jnp.float32),
        pltpu.VMEM((SCH, HALF), jnp.float32),
        pltpu.VMEM((SCH, HALF), jnp.float32),
        pltpu.VMEM((80, HALF), jnp.float32),
        pltpu.SemaphoreType.DMA,
        pltpu.SemaphoreType.DMA,
    ]


def _spmm_x_call(y, src2d, dst2d, zeros80, zdrain):
    mesh = plsc.VectorSubcoreMesh(core_axis_name="c", subcore_axis_name="s")
    return pl.kernel(
        _spmm_x_body,
        out_type=jax.ShapeDtypeStruct((NC, NPAD, HALF), jnp.float32),
        mesh=mesh,
        scratch_types=_spmm_scratch(),
    )(y, src2d, dst2d, zeros80, zdrain)


def _spmm_call(s_pair, src2d, dst2d, zeros80, zdrain):
    mesh = plsc.VectorSubcoreMesh(core_axis_name="c", subcore_axis_name="s")
    return pl.kernel(
        _spmm_body,
        out_type=jax.ShapeDtypeStruct((NC, NPAD, HALF), jnp.float32),
        mesh=mesh,
        scratch_types=_spmm_scratch(),
    )(s_pair, src2d, dst2d, zeros80, zdrain)


def _tca_body(x_ref, deg_ref, y_ref, dinv_ref):
    deg = deg_ref[0, :, 0:1] + deg_ref[1, :, 0:1] + 1.0
    dinv = lax.rsqrt(deg)
    y_ref[...] = x_ref[...] * dinv
    dinv_ref[...] = dinv


def _tca_call(xp, deg2):
    return pl.pallas_call(
        _tca_body,
        grid=(GRID,),
        in_specs=[
            pl.BlockSpec((RB, D_IN), lambda i: (i, 0)),
            pl.BlockSpec((NC, RB, HALF), lambda i: (0, i, 0)),
        ],
        out_specs=[
            pl.BlockSpec((RB, D_IN), lambda i: (i, 0)),
            pl.BlockSpec((RB, 1), lambda i: (i, 0)),
        ],
        out_shape=[
            jax.ShapeDtypeStruct((NPAD, D_IN), jnp.float32),
            jax.ShapeDtypeStruct((NPAD, 1), jnp.float32),
        ],
    )(xp, deg2)


def _tcb_body(aggx_ref, y_ref, dinv_ref, b1_ref, w1_ref, w2_ref, out_ref):
    dinv = dinv_ref[...]
    u = aggx_ref[0] + aggx_ref[1] + y_ref[...]
    z1 = jnp.dot(u, w1_ref[...], preferred_element_type=jnp.float32)
    h1 = jnp.maximum(z1 * dinv + b1_ref[...], 0.0)
    z2 = jnp.dot(h1, w2_ref[...], preferred_element_type=jnp.float32)
    s2 = z2 * dinv
    out_ref[0] = s2[:, :HALF]
    out_ref[1] = s2[:, HALF:]


def _tcb_call(aggx, y, dinv, b1r, W1, W2):
    return pl.pallas_call(
        _tcb_body,
        grid=(GRID,),
        in_specs=[
            pl.BlockSpec((NC, RB, HALF), lambda i: (0, i, 0)),
            pl.BlockSpec((RB, D_IN), lambda i: (i, 0)),
            pl.BlockSpec((RB, 1), lambda i: (i, 0)),
            pl.BlockSpec((1, D_HID), lambda i: (0, 0)),
            pl.BlockSpec((D_IN, D_HID), lambda i: (0, 0)),
            pl.BlockSpec((D_HID, D_HID), lambda i: (0, 0)),
        ],
        out_specs=pl.BlockSpec((NC, RB, HALF), lambda i: (0, i, 0)),
        out_shape=jax.ShapeDtypeStruct((NC, NPAD, HALF), jnp.float32),
    )(aggx, y, dinv, b1r, W1, W2)


def _tcc_body(agg_ref, s_ref, dinv_ref, b2_ref, batch_ref, cf_ref,
              lwg_ref, lwc_ref, lb_ref, out_ref, acc_sum, acc_cnt):
    i = pl.program_id(0)

    @pl.when(i == 0)
    def _():
        acc_sum[...] = jnp.zeros_like(acc_sum)
        acc_cnt[...] = jnp.zeros_like(acc_cnt)

    dinv = dinv_ref[...]
    h2a = jnp.maximum((agg_ref[0] + s_ref[0]) * dinv + b2_ref[:, :HALF], 0.0)
    h2b = jnp.maximum((agg_ref[1] + s_ref[1]) * dinv + b2_ref[:, HALF:], 0.0)
    h2 = jnp.concatenate([h2a, h2b], axis=1)
    bb = batch_ref[...]
    ids = lax.broadcasted_iota(jnp.int32, (RB, B), 1)
    pt = (bb == ids).astype(jnp.float32)
    acc_sum[...] += lax.dot_general(pt, h2, (((0,), (0,)), ((), ())),
                                    preferred_element_type=jnp.float32)
    acc_cnt[...] += lax.dot_general(pt, jnp.ones((RB, 1), jnp.float32),
                                    (((0,), (0,)), ((), ())),
                                    preferred_element_type=jnp.float32)

    @pl.when(i == pl.num_programs(0) - 1)
    def _():
        g = acc_sum[...] / jnp.maximum(acc_cnt[...], 1.0)
        res = (jnp.dot(g, lwg_ref[...], preferred_element_type=jnp.float32)
               + jnp.dot(cf_ref[...], lwc_ref[...],
                         preferred_element_type=jnp.float32)
               + lb_ref[...])
        out_ref[...] = res


def _tcc_call(agg2, s2_pair, dinv, b2r, batch_p, cf, lwg, lwc, lbr):
    return pl.pallas_call(
        _tcc_body,
        grid=(GRID,),
        in_specs=[
            pl.BlockSpec((NC, RB, HALF), lambda i: (0, i, 0)),
            pl.BlockSpec((NC, RB, HALF), lambda i: (0, i, 0)),
            pl.BlockSpec((RB, 1), lambda i: (i, 0)),
            pl.BlockSpec((1, D_HID), lambda i: (0, 0)),
            pl.BlockSpec((RB, 1), lambda i: (i, 0)),
            pl.BlockSpec((B, EXTRA), lambda i: (0, 0)),
            pl.BlockSpec((D_HID, 2), lambda i: (0, 0)),
            pl.BlockSpec((EXTRA, 2), lambda i: (0, 0)),
            pl.BlockSpec((1, 2), lambda i: (0, 0)),
        ],
        out_specs=pl.BlockSpec((B, 2), lambda i: (0, 0)),
        out_shape=jax.ShapeDtypeStruct((B, 2), jnp.float32),
        scratch_shapes=[
            pltpu.VMEM((B, D_HID), jnp.float32),
            pltpu.VMEM((B, 1), jnp.float32),
        ],
    )(agg2, s2_pair, dinv, b2r, batch_p, cf, lwg, lwc, lbr)


def kernel(x, edge_index, batch, cluster_feats, W1, b1, W2, b2, lin_W, lin_b):
    f32 = jnp.float32
    src2d = edge_index[0].reshape(SROWS, SCH)
    dst2d = edge_index[1].reshape(SROWS, SCH)
    dst2d_deg = edge_index[1].reshape(ROWS_E, CH)
    xp = jnp.zeros((NPAD, D_IN), f32).at[:N].set(x)
    batch_p = jnp.concatenate(
        [batch, jnp.full((NPAD - N,), B, jnp.int32)]).reshape(NPAD, 1)
    ones128 = jnp.ones((CH, HALF), f32)
    zeros80 = jnp.zeros((80, HALF), f32)
    zdrain = jnp.zeros((SCH, HALF), f32)
    b1r = b1.reshape(1, D_HID)
    b2r = b2.reshape(1, D_HID)
    lwg = lin_W[:D_HID]
    lwc = lin_W[D_HID:]
    lbr = lin_b.reshape(1, 2)

    deg2 = _deg_call(dst2d_deg, zeros80, ones128)
    y, dinv = _tca_call(xp, deg2)
    aggx = _spmm_x_call(y, src2d, dst2d, zeros80, zdrain)
    s2_pair = _tcb_call(aggx, y, dinv, b1r, W1, W2)
    agg2 = _spmm_call(s2_pair, src2d, dst2d, zeros80, zdrain)
    return _tcc_call(agg2, s2_pair, dinv, b2r, batch_p, cluster_feats,
                     lwg, lwc, lbr)
